# baseline (device time: 32555 ns/iter reference)
import jax
import jax.numpy as jnp
from jax import lax
from jax.experimental import pallas as pl
from jax.experimental.pallas import tpu as pltpu

N_DEV = 4
B_LOC = 2
SQ = 128
SKV = 128
D_MODEL = 512
H_LOC = 4
H_GLOB = 16
DH = 64
DH_LOC = H_LOC * DH


def kernel(x, Wq, K_ext, V_ext, Wo):
    def body(x_ref, wq_ref, k_ref, v_ref, wo_ref, out_ref,
             wq_all, wo_all, k_all, v_all,
             send_sems, recv_sems, kv_sems):
        my = lax.axis_index("i")

        kv_copies = []
        for hg in range(H_GLOB):
            ck = pltpu.make_async_copy(
                k_ref.at[pl.ds(my * B_LOC, B_LOC), :, hg, :],
                k_all.at[hg], kv_sems.at[0, hg])
            cv = pltpu.make_async_copy(
                v_ref.at[pl.ds(my * B_LOC, B_LOC), :, hg, :],
                v_all.at[hg], kv_sems.at[1, hg])
            ck.start()
            cv.start()
            kv_copies.append((ck, cv))

        barrier = pltpu.get_barrier_semaphore()
        for off in range(1, N_DEV):
            pl.semaphore_signal(
                barrier, inc=1,
                device_id=((my + off) % N_DEV,),
                device_id_type=pl.DeviceIdType.MESH,
            )
        pl.semaphore_wait(barrier, N_DEV - 1)

        wq_all[0] = wq_ref[...].astype(jnp.bfloat16)
        wo_all[0] = wo_ref[...].astype(jnp.bfloat16)

        rdmas = {}
        for off in range(1, N_DEV):
            for t, buf in ((0, wq_all), (1, wo_all)):
                r = pltpu.make_async_remote_copy(
                    src_ref=buf.at[0],
                    dst_ref=buf.at[off],
                    send_sem=send_sems.at[t, off],
                    recv_sem=recv_sems.at[t, off],
                    device_id=((my + off) % N_DEV,),
                    device_id_type=pl.DeviceIdType.MESH,
                )
                r.start()
                rdmas[(t, off)] = r

        x2 = x_ref[...].reshape(B_LOC * SQ, D_MODEL).astype(jnp.bfloat16)

        qi = lax.broadcasted_iota(jnp.int32, (SQ, SKV), 0)
        kj = lax.broadcasted_iota(jnp.int32, (SQ, SKV), 1)
        mask_add = jnp.where((qi < 64) & (kj >= 64),
                             jnp.float32(-1e9), jnp.float32(0.0))

        def group_out(slot, g):
            wq_g = wq_all[slot]
            wo_g = wo_all[slot].reshape(H_LOC, DH, D_MODEL)
            q = jnp.stack(
                [jnp.dot(x2, wq_g[:, h * DH:(h + 1) * DH],
                         preferred_element_type=jnp.float32)
                 .astype(jnp.bfloat16).reshape(B_LOC, SQ, DH)
                 for h in range(H_LOC)],
                axis=0).reshape(H_LOC * B_LOC, SQ, DH)
            kg = (k_all[pl.ds(g * H_LOC, H_LOC)].astype(jnp.bfloat16)
                  .reshape(H_LOC * B_LOC, SKV, DH))
            vg = (v_all[pl.ds(g * H_LOC, H_LOC)].astype(jnp.bfloat16)
                  .reshape(H_LOC * B_LOC, SKV, DH))
            scores = lax.dot_general(
                q, kg, (((2,), (2,)), ((0,), (0,))),
                preferred_element_type=jnp.float32)
            scores = scores * jnp.float32(0.125) + mask_add[None]
            m = jnp.max(scores, axis=-1, keepdims=True)
            w = jnp.exp(scores - m)
            w = w / jnp.sum(w, axis=-1, keepdims=True)
            ctx = lax.dot_general(
                w.astype(jnp.bfloat16), vg, (((2,), (1,)), ((0,), (0,))),
                preferred_element_type=jnp.float32)
            ctx = ctx.astype(jnp.bfloat16).reshape(H_LOC, B_LOC * SQ, DH)
            acc = None
            for h in range(H_LOC):
                part = jnp.dot(ctx[h], wo_g[h],
                               preferred_element_type=jnp.float32)
                acc = part if acc is None else acc + part
            return acc

        for ck, cv in kv_copies:
            ck.wait()
            cv.wait()

        out = group_out(0, my)
        for off in (1, 3, 2):
            rdmas[(0, off)].wait_recv()
            rdmas[(1, off)].wait_recv()
            out = out + group_out(off, (my - off) % N_DEV)

        for r in rdmas.values():
            r.wait_send()

        out_ref[...] = out.reshape(B_LOC, SQ, D_MODEL)

    return pl.pallas_call(
        body,
        out_shape=jax.ShapeDtypeStruct((B_LOC, SQ, D_MODEL), jnp.float32),
        in_specs=[
            pl.BlockSpec(memory_space=pltpu.VMEM),
            pl.BlockSpec(memory_space=pltpu.VMEM),
            pl.BlockSpec(memory_space=pl.ANY),
            pl.BlockSpec(memory_space=pl.ANY),
            pl.BlockSpec(memory_space=pltpu.VMEM),
        ],
        out_specs=pl.BlockSpec(memory_space=pltpu.VMEM),
        scratch_shapes=[
            pltpu.VMEM((N_DEV, D_MODEL, DH_LOC), jnp.bfloat16),
            pltpu.VMEM((N_DEV, DH_LOC, D_MODEL), jnp.bfloat16),
            pltpu.VMEM((H_GLOB, B_LOC, SKV, DH), jnp.float32),
            pltpu.VMEM((H_GLOB, B_LOC, SKV, DH), jnp.float32),
            pltpu.SemaphoreType.DMA((2, N_DEV)),
            pltpu.SemaphoreType.DMA((2, N_DEV)),
            pltpu.SemaphoreType.DMA((2, H_GLOB)),
        ],
        compiler_params=pltpu.CompilerParams(collective_id=0),
    )(x, Wq, K_ext, V_ext, Wo)


# device time: 26669 ns/iter; 1.2207x vs baseline; 1.2207x over previous
import os

import jax
import jax.numpy as jnp
from jax import lax
from jax.experimental import pallas as pl
from jax.experimental.pallas import tpu as pltpu

_VARIANT = os.environ.get("KVARIANT", "full")
_COMM = _VARIANT in ("full", "nocomp")

N_DEV = 4
B_LOC = 2
SQ = 128
SKV = 128
D_MODEL = 512
H_LOC = 4
H_GLOB = 16
DH = 64
DH_LOC = H_LOC * DH


def kernel(x, Wq, K_ext, V_ext, Wo):
    i = lax.axis_index("i")
    K_loc = lax.dynamic_slice_in_dim(K_ext, i * B_LOC, B_LOC, axis=0)
    V_loc = lax.dynamic_slice_in_dim(V_ext, i * B_LOC, B_LOC, axis=0)

    def body(x_ref, wq_ref, k_ref, v_ref, wo_ref, out_ref,
             wq_all, wo_all, k_all, v_all,
             send_sems, recv_sems, kv_sems):
        my = lax.axis_index("i")

        if _VARIANT == "noop":
            out_ref[...] = x_ref[...].astype(jnp.float32)
            return

        kv_copies = []
        for hg in range(H_GLOB):
            ck = pltpu.make_async_copy(
                k_ref.at[:, :, hg, :], k_all.at[hg], kv_sems.at[0, hg])
            cv = pltpu.make_async_copy(
                v_ref.at[:, :, hg, :], v_all.at[hg], kv_sems.at[1, hg])
            ck.start()
            cv.start()
            kv_copies.append((ck, cv))

        if _COMM:
            barrier = pltpu.get_barrier_semaphore()
            for off in range(1, N_DEV):
                pl.semaphore_signal(
                    barrier, inc=1,
                    device_id=((my + off) % N_DEV,),
                    device_id_type=pl.DeviceIdType.MESH,
                )
            pl.semaphore_wait(barrier, N_DEV - 1)

        wq_all[0] = wq_ref[...].astype(jnp.bfloat16)
        wo_all[0] = wo_ref[...].astype(jnp.bfloat16)

        rdmas = {}
        if _COMM:
            for off in range(1, N_DEV):
                for t, buf in ((0, wq_all), (1, wo_all)):
                    r = pltpu.make_async_remote_copy(
                        src_ref=buf.at[0],
                        dst_ref=buf.at[off],
                        send_sem=send_sems.at[t, off],
                        recv_sem=recv_sems.at[t, off],
                        device_id=((my + off) % N_DEV,),
                        device_id_type=pl.DeviceIdType.MESH,
                    )
                    r.start()
                    rdmas[(t, off)] = r

        x2 = x_ref[...].reshape(B_LOC * SQ, D_MODEL).astype(jnp.bfloat16)

        qi = lax.broadcasted_iota(jnp.int32, (SQ, SKV), 0)
        kj = lax.broadcasted_iota(jnp.int32, (SQ, SKV), 1)
        mask_add = jnp.where((qi < 64) & (kj >= 64),
                             jnp.float32(-1e9), jnp.float32(0.0))

        def group_out(slot, g):
            wq_g = wq_all[slot]
            wo_g = wo_all[slot].reshape(H_LOC, DH, D_MODEL)
            q = jnp.stack(
                [jnp.dot(x2, wq_g[:, h * DH:(h + 1) * DH],
                         preferred_element_type=jnp.float32)
                 .astype(jnp.bfloat16).reshape(B_LOC, SQ, DH)
                 for h in range(H_LOC)],
                axis=0).reshape(H_LOC * B_LOC, SQ, DH)
            kg = (k_all[pl.ds(g * H_LOC, H_LOC)].astype(jnp.bfloat16)
                  .reshape(H_LOC * B_LOC, SKV, DH))
            vg = (v_all[pl.ds(g * H_LOC, H_LOC)].astype(jnp.bfloat16)
                  .reshape(H_LOC * B_LOC, SKV, DH))
            scores = lax.dot_general(
                q, kg, (((2,), (2,)), ((0,), (0,))),
                preferred_element_type=jnp.float32)
            scores = scores * jnp.float32(0.125) + mask_add[None]
            m = jnp.max(scores, axis=-1, keepdims=True)
            w = jnp.exp(scores - m)
            w = w / jnp.sum(w, axis=-1, keepdims=True)
            ctx = lax.dot_general(
                w.astype(jnp.bfloat16), vg, (((2,), (1,)), ((0,), (0,))),
                preferred_element_type=jnp.float32)
            ctx = ctx.astype(jnp.bfloat16).reshape(H_LOC, B_LOC * SQ, DH)
            acc = None
            for h in range(H_LOC):
                part = jnp.dot(ctx[h], wo_g[h],
                               preferred_element_type=jnp.float32)
                acc = part if acc is None else acc + part
            return acc

        for ck, cv in kv_copies:
            ck.wait()
            cv.wait()

        if _VARIANT == "nocomp":
            for off in (1, 3, 2):
                rdmas[(0, off)].wait_recv()
                rdmas[(1, off)].wait_recv()
            for r in rdmas.values():
                r.wait_send()
            out_ref[...] = (x_ref[...].astype(jnp.float32)
                            + jnp.sum(wq_all[3].astype(jnp.float32)))
            return

        if _VARIANT == "kvonly":
            out_ref[...] = (x_ref[...].astype(jnp.float32)
                            + jnp.sum(k_all[0].astype(jnp.float32))
                            + jnp.sum(v_all[15].astype(jnp.float32)))
            return

        if _VARIANT == "one_group":
            out_ref[...] = (group_out(0, my) * 4.0).reshape(B_LOC, SQ, D_MODEL)
            return

        out = group_out(0, my)
        for off in (1, 3, 2):
            if _COMM:
                rdmas[(0, off)].wait_recv()
                rdmas[(1, off)].wait_recv()
            slot = off if _COMM else 0
            out = out + group_out(slot, (my - off) % N_DEV)

        for r in rdmas.values():
            r.wait_send()

        out_ref[...] = out.reshape(B_LOC, SQ, D_MODEL)

    return pl.pallas_call(
        body,
        out_shape=jax.ShapeDtypeStruct((B_LOC, SQ, D_MODEL), jnp.float32),
        in_specs=[
            pl.BlockSpec(memory_space=pltpu.VMEM),
            pl.BlockSpec(memory_space=pltpu.VMEM),
            pl.BlockSpec(memory_space=pltpu.VMEM),
            pl.BlockSpec(memory_space=pltpu.VMEM),
            pl.BlockSpec(memory_space=pltpu.VMEM),
        ],
        out_specs=pl.BlockSpec(memory_space=pltpu.VMEM),
        scratch_shapes=[
            pltpu.VMEM((N_DEV, D_MODEL, DH_LOC), jnp.bfloat16),
            pltpu.VMEM((N_DEV, DH_LOC, D_MODEL), jnp.bfloat16),
            pltpu.VMEM((H_GLOB, B_LOC, SKV, DH), jnp.float32),
            pltpu.VMEM((H_GLOB, B_LOC, SKV, DH), jnp.float32),
            pltpu.SemaphoreType.DMA((2, N_DEV)),
            pltpu.SemaphoreType.DMA((2, N_DEV)),
            pltpu.SemaphoreType.DMA((2, H_GLOB)),
        ],
        compiler_params=(
            pltpu.CompilerParams(collective_id=0)
            if _COMM
            else pltpu.CompilerParams()
        ),
    )(x, Wq, K_loc, V_loc, Wo)


# device time: 22605 ns/iter; 1.4402x vs baseline; 1.1798x over previous
import os

import jax
import jax.numpy as jnp
from jax import lax
from jax.experimental import pallas as pl
from jax.experimental.pallas import tpu as pltpu

_VARIANT = os.environ.get("KVARIANT", "full")
_COMM = _VARIANT in ("full", "nocomp")

N_DEV = 4
B_LOC = 2
SQ = 128
SKV = 128
D_MODEL = 512
H_LOC = 4
H_GLOB = 16
DH = 64
DH_LOC = H_LOC * DH


def kernel(x, Wq, K_ext, V_ext, Wo):
    i = lax.axis_index("i")
    K_loc = lax.dynamic_slice_in_dim(K_ext, i * B_LOC, B_LOC, axis=0)
    V_loc = lax.dynamic_slice_in_dim(V_ext, i * B_LOC, B_LOC, axis=0)

    def body(x_ref, wq_ref, k_ref, v_ref, wo_ref, out_ref,
             wq_all, wo_all, k_all, v_all,
             send_sems, recv_sems, kv_sems):
        my = lax.axis_index("i")

        if _VARIANT == "noop":
            out_ref[...] = x_ref[...].astype(jnp.float32)
            return

        kv_copies = []
        for hg in range(H_GLOB):
            ck = pltpu.make_async_copy(
                k_ref.at[:, :, hg, :], k_all.at[hg], kv_sems.at[0, hg])
            cv = pltpu.make_async_copy(
                v_ref.at[:, :, hg, :], v_all.at[hg], kv_sems.at[1, hg])
            ck.start()
            cv.start()
            kv_copies.append((ck, cv))

        right = (my + 1) % N_DEV
        left = (my - 1) % N_DEV
        if _COMM:
            barrier = pltpu.get_barrier_semaphore()
            for nbr in (left, right):
                pl.semaphore_signal(
                    barrier, inc=1,
                    device_id=(nbr,), device_id_type=pl.DeviceIdType.MESH,
                )
            pl.semaphore_wait(barrier, 2)

        wq_all[0] = wq_ref[...].astype(jnp.bfloat16)
        wo_all[0] = wo_ref[...].astype(jnp.bfloat16)

        def rdma(t, buf, src_slot, dst_slot, dev):
            return pltpu.make_async_remote_copy(
                src_ref=buf.at[src_slot],
                dst_ref=buf.at[dst_slot],
                send_sem=send_sems.at[t, dst_slot],
                recv_sem=recv_sems.at[t, dst_slot],
                device_id=(dev,),
                device_id_type=pl.DeviceIdType.MESH,
            )

        rdmas = {}
        if _COMM:
            rdmas[(0, 1)] = rdma(0, wq_all, 0, 1, right)
            rdmas[(1, 1)] = rdma(1, wo_all, 0, 1, right)
            rdmas[(1, 3)] = rdma(1, wo_all, 0, 3, left)
            rdmas[(0, 3)] = rdma(0, wq_all, 0, 3, left)
            rdmas[(0, 1)].start()
            rdmas[(1, 1)].start()
            rdmas[(1, 3)].start()
            rdmas[(0, 3)].start()
            rdmas[(0, 2)] = rdma(0, wq_all, 1, 2, right)
            rdmas[(1, 2)] = rdma(1, wo_all, 3, 2, left)

        x2 = x_ref[...].reshape(B_LOC * SQ, D_MODEL).astype(jnp.bfloat16)

        qi = lax.broadcasted_iota(jnp.int32, (SQ, SKV), 0)
        kj = lax.broadcasted_iota(jnp.int32, (SQ, SKV), 1)
        mask_add = jnp.where((qi < 64) & (kj >= 64),
                             jnp.float32(-1e9), jnp.float32(0.0))

        def group_out(slot, g):
            wq_g = wq_all[slot]
            wo_g = wo_all[slot].reshape(H_LOC, DH, D_MODEL)
            q = jnp.stack(
                [jnp.dot(x2, wq_g[:, h * DH:(h + 1) * DH],
                         preferred_element_type=jnp.float32)
                 .astype(jnp.bfloat16).reshape(B_LOC, SQ, DH)
                 for h in range(H_LOC)],
                axis=0).reshape(H_LOC * B_LOC, SQ, DH)
            kg = (k_all[pl.ds(g * H_LOC, H_LOC)].astype(jnp.bfloat16)
                  .reshape(H_LOC * B_LOC, SKV, DH))
            vg = (v_all[pl.ds(g * H_LOC, H_LOC)].astype(jnp.bfloat16)
                  .reshape(H_LOC * B_LOC, SKV, DH))
            scores = lax.dot_general(
                q, kg, (((2,), (2,)), ((0,), (0,))),
                preferred_element_type=jnp.float32)
            scores = scores * jnp.float32(0.125) + mask_add[None]
            m = jnp.max(scores, axis=-1, keepdims=True)
            w = jnp.exp(scores - m)
            w = w / jnp.sum(w, axis=-1, keepdims=True)
            ctx = lax.dot_general(
                w.astype(jnp.bfloat16), vg, (((2,), (1,)), ((0,), (0,))),
                preferred_element_type=jnp.float32)
            ctx = ctx.astype(jnp.bfloat16).reshape(H_LOC, B_LOC * SQ, DH)
            acc = None
            for h in range(H_LOC):
                part = jnp.dot(ctx[h], wo_g[h],
                               preferred_element_type=jnp.float32)
                acc = part if acc is None else acc + part
            return acc

        for ck, cv in kv_copies:
            ck.wait()
            cv.wait()

        if _VARIANT == "nocomp":
            rdmas[(0, 1)].wait_recv()
            rdmas[(0, 2)].start()
            rdmas[(1, 3)].wait_recv()
            rdmas[(1, 2)].start()
            rdmas[(1, 1)].wait_recv()
            rdmas[(0, 3)].wait_recv()
            rdmas[(0, 2)].wait_recv()
            rdmas[(1, 2)].wait_recv()
            for r in rdmas.values():
                r.wait_send()
            out_ref[...] = (x_ref[...].astype(jnp.float32)
                            + jnp.sum(wq_all[3].astype(jnp.float32)))
            return

        if _VARIANT == "kvonly":
            out_ref[...] = (x_ref[...].astype(jnp.float32)
                            + jnp.sum(k_all[0].astype(jnp.float32))
                            + jnp.sum(v_all[15].astype(jnp.float32)))
            return

        if _VARIANT == "one_group":
            out_ref[...] = (group_out(0, my) * 4.0).reshape(B_LOC, SQ, D_MODEL)
            return

        out = group_out(0, my)
        if _COMM:
            rdmas[(0, 1)].wait_recv()
            rdmas[(0, 2)].start()
            rdmas[(1, 3)].wait_recv()
            rdmas[(1, 2)].start()
            rdmas[(1, 1)].wait_recv()
            out = out + group_out(1, left)
            rdmas[(0, 3)].wait_recv()
            out = out + group_out(3, right)
            rdmas[(0, 2)].wait_recv()
            rdmas[(1, 2)].wait_recv()
            out = out + group_out(2, (my + 2) % N_DEV)
            for r in rdmas.values():
                r.wait_send()
        else:
            for off in (1, 3, 2):
                out = out + group_out(0, (my - off) % N_DEV)

        out_ref[...] = out.reshape(B_LOC, SQ, D_MODEL)

    return pl.pallas_call(
        body,
        out_shape=jax.ShapeDtypeStruct((B_LOC, SQ, D_MODEL), jnp.float32),
        in_specs=[
            pl.BlockSpec(memory_space=pltpu.VMEM),
            pl.BlockSpec(memory_space=pltpu.VMEM),
            pl.BlockSpec(memory_space=pltpu.VMEM),
            pl.BlockSpec(memory_space=pltpu.VMEM),
            pl.BlockSpec(memory_space=pltpu.VMEM),
        ],
        out_specs=pl.BlockSpec(memory_space=pltpu.VMEM),
        scratch_shapes=[
            pltpu.VMEM((N_DEV, D_MODEL, DH_LOC), jnp.bfloat16),
            pltpu.VMEM((N_DEV, DH_LOC, D_MODEL), jnp.bfloat16),
            pltpu.VMEM((H_GLOB, B_LOC, SKV, DH), jnp.float32),
            pltpu.VMEM((H_GLOB, B_LOC, SKV, DH), jnp.float32),
            pltpu.SemaphoreType.DMA((2, N_DEV)),
            pltpu.SemaphoreType.DMA((2, N_DEV)),
            pltpu.SemaphoreType.DMA((2, H_GLOB)),
        ],
        compiler_params=(
            pltpu.CompilerParams(collective_id=0)
            if _COMM
            else pltpu.CompilerParams()
        ),
    )(x, Wq, K_loc, V_loc, Wo)


# device time: 21783 ns/iter; 1.4945x vs baseline; 1.0377x over previous
import os

import jax
import jax.numpy as jnp
from jax import lax
from jax.experimental import pallas as pl
from jax.experimental.pallas import tpu as pltpu

_VARIANT = os.environ.get("KVARIANT", "full")
_COMM = _VARIANT in ("full", "nocomp")

N_DEV = 4
B_LOC = 2
SQ = 128
SKV = 128
D_MODEL = 512
H_LOC = 4
H_GLOB = 16
DH = 64
DH_LOC = H_LOC * DH


def kernel(x, Wq, K_ext, V_ext, Wo):
    i = lax.axis_index("i")
    K_loc = jnp.transpose(
        lax.dynamic_slice_in_dim(K_ext, i * B_LOC, B_LOC, axis=0),
        (2, 0, 1, 3)).astype(jnp.bfloat16)
    V_loc = jnp.transpose(
        lax.dynamic_slice_in_dim(V_ext, i * B_LOC, B_LOC, axis=0),
        (2, 0, 1, 3)).astype(jnp.bfloat16)
    x = x.astype(jnp.bfloat16)
    Wq = Wq.astype(jnp.bfloat16)
    Wo = Wo.astype(jnp.bfloat16)

    def body(x_ref, wq_ref, k_ref, v_ref, wo_ref, out_ref,
             wq_all, wo_all, send_sems, recv_sems):
        my = lax.axis_index("i")

        if _VARIANT == "noop":
            out_ref[...] = x_ref[...].astype(jnp.float32)
            return

        right = (my + 1) % N_DEV
        left = (my - 1) % N_DEV
        if _COMM:
            barrier = pltpu.get_barrier_semaphore()
            for nbr in (left, right):
                pl.semaphore_signal(
                    barrier, inc=1,
                    device_id=(nbr,), device_id_type=pl.DeviceIdType.MESH,
                )
            pl.semaphore_wait(barrier, 2)

        wq_all[0] = wq_ref[...]
        wo_all[0] = wo_ref[...]

        def rdma(t, buf, src_slot, dst_slot, dev):
            return pltpu.make_async_remote_copy(
                src_ref=buf.at[src_slot],
                dst_ref=buf.at[dst_slot],
                send_sem=send_sems.at[t, dst_slot],
                recv_sem=recv_sems.at[t, dst_slot],
                device_id=(dev,),
                device_id_type=pl.DeviceIdType.MESH,
            )

        rdmas = {}
        if _COMM:
            rdmas[(0, 1)] = rdma(0, wq_all, 0, 1, right)
            rdmas[(1, 1)] = rdma(1, wo_all, 0, 1, right)
            rdmas[(1, 3)] = rdma(1, wo_all, 0, 3, left)
            rdmas[(0, 3)] = rdma(0, wq_all, 0, 3, left)
            rdmas[(0, 1)].start()
            rdmas[(1, 1)].start()
            rdmas[(1, 3)].start()
            rdmas[(0, 3)].start()
            rdmas[(0, 2)] = rdma(0, wq_all, 1, 2, right)
            rdmas[(1, 2)] = rdma(1, wo_all, 3, 2, left)

        x2 = x_ref[...].reshape(B_LOC * SQ, D_MODEL)

        qi = lax.broadcasted_iota(jnp.int32, (SQ, SKV), 0)
        kj = lax.broadcasted_iota(jnp.int32, (SQ, SKV), 1)
        mask_add = jnp.where((qi < 64) & (kj >= 64),
                             jnp.float32(-1e9), jnp.float32(0.0))

        def group_out(slot, g):
            wq_g = wq_all[slot]
            wo_g = wo_all[slot].reshape(H_LOC, DH, D_MODEL)
            q = jnp.stack(
                [jnp.dot(x2, wq_g[:, h * DH:(h + 1) * DH],
                         preferred_element_type=jnp.float32)
                 .astype(jnp.bfloat16).reshape(B_LOC, SQ, DH)
                 for h in range(H_LOC)],
                axis=0).reshape(H_LOC * B_LOC, SQ, DH)
            kg = k_ref[pl.ds(g * H_LOC, H_LOC)].reshape(H_LOC * B_LOC, SKV, DH)
            vg = v_ref[pl.ds(g * H_LOC, H_LOC)].reshape(H_LOC * B_LOC, SKV, DH)
            scores = lax.dot_general(
                q, kg, (((2,), (2,)), ((0,), (0,))),
                preferred_element_type=jnp.float32)
            scores = scores * jnp.float32(0.125) + mask_add[None]
            m = jnp.max(scores, axis=-1, keepdims=True)
            w = jnp.exp(scores - m)
            w = w / jnp.sum(w, axis=-1, keepdims=True)
            ctx = lax.dot_general(
                w.astype(jnp.bfloat16), vg, (((2,), (1,)), ((0,), (0,))),
                preferred_element_type=jnp.float32)
            ctx = ctx.astype(jnp.bfloat16).reshape(H_LOC, B_LOC * SQ, DH)
            acc = None
            for h in range(H_LOC):
                part = jnp.dot(ctx[h], wo_g[h],
                               preferred_element_type=jnp.float32)
                acc = part if acc is None else acc + part
            return acc

        if _VARIANT == "nocomp":
            rdmas[(0, 1)].wait_recv()
            rdmas[(0, 2)].start()
            rdmas[(1, 3)].wait_recv()
            rdmas[(1, 2)].start()
            rdmas[(1, 1)].wait_recv()
            rdmas[(0, 3)].wait_recv()
            rdmas[(0, 2)].wait_recv()
            rdmas[(1, 2)].wait_recv()
            for r in rdmas.values():
                r.wait_send()
            out_ref[...] = (x_ref[...].astype(jnp.float32)
                            + jnp.sum(wq_all[3].astype(jnp.float32)))
            return

        if _VARIANT == "one_group":
            out_ref[...] = (group_out(0, my) * 4.0).reshape(B_LOC, SQ, D_MODEL)
            return

        out = group_out(0, my)
        if _COMM:
            rdmas[(0, 1)].wait_recv()
            rdmas[(0, 2)].start()
            rdmas[(1, 3)].wait_recv()
            rdmas[(1, 2)].start()
            rdmas[(1, 1)].wait_recv()
            out = out + group_out(1, left)
            rdmas[(0, 3)].wait_recv()
            out = out + group_out(3, right)
            rdmas[(0, 2)].wait_recv()
            rdmas[(1, 2)].wait_recv()
            out = out + group_out(2, (my + 2) % N_DEV)
            for r in rdmas.values():
                r.wait_send()
        else:
            for off in (1, 3, 2):
                out = out + group_out(0, (my - off) % N_DEV)

        out_ref[...] = out.reshape(B_LOC, SQ, D_MODEL)

    return pl.pallas_call(
        body,
        out_shape=jax.ShapeDtypeStruct((B_LOC, SQ, D_MODEL), jnp.float32),
        in_specs=[
            pl.BlockSpec(memory_space=pltpu.VMEM),
            pl.BlockSpec(memory_space=pltpu.VMEM),
            pl.BlockSpec(memory_space=pltpu.VMEM),
            pl.BlockSpec(memory_space=pltpu.VMEM),
            pl.BlockSpec(memory_space=pltpu.VMEM),
        ],
        out_specs=pl.BlockSpec(memory_space=pltpu.VMEM),
        scratch_shapes=[
            pltpu.VMEM((N_DEV, D_MODEL, DH_LOC), jnp.bfloat16),
            pltpu.VMEM((N_DEV, DH_LOC, D_MODEL), jnp.bfloat16),
            pltpu.SemaphoreType.DMA((2, N_DEV)),
            pltpu.SemaphoreType.DMA((2, N_DEV)),
        ],
        compiler_params=(
            pltpu.CompilerParams(collective_id=0)
            if _COMM
            else pltpu.CompilerParams()
        ),
    )(x, Wq, K_loc, V_loc, Wo)
